# baseline (speedup 1.0000x reference)
import functools

import jax
import jax.numpy as jnp
from jax import lax
from jax.experimental import pallas as pl
from jax.experimental.pallas import tpu as pltpu
from jax.experimental.pallas import tpu_sc as plsc

N = 10000
E = 320000
B = 64
DF = 128
H = 64

NC = 2
NS = 16
HP = 64
CH = 128
EPT = E // NS
NCHUNK = 160
EPTP = NCHUNK * CH
NBUF = 4
NWIN = 2
WCH = NCHUNK // NWIN
NITER = WCH // NBUF
RA = 632
RB = N - (NS - 1) * RA

BR = 2000
NBLK = N // BR


def _sc_scatter_body(table, src, dst, out, sidx, didx,
                     rows, gsems, ssem, acc):
    c = lax.axis_index("c")
    s = lax.axis_index("s")
    gbase = c * N
    rbase = s * RA

    @pl.when(s < NS - 1)
    def _():
        pltpu.sync_copy(table.at[pl.ds(gbase + rbase, RA)], acc.at[pl.ds(rbase, RA)])

    @pl.when(s == NS - 1)
    def _():
        pltpu.sync_copy(table.at[pl.ds(gbase + rbase, RB)], acc.at[pl.ds(rbase, RB)])

    plsc.subcore_barrier()

    pltpu.sync_copy(src.at[c, s], sidx)
    pltpu.sync_copy(dst.at[c, s], didx)

    def chunk(j, carry):
        pltpu.async_copy(table.at[sidx.at[j]], rows[0], gsems[0]).wait()
        pltpu.sync_copy(rows[0], acc.at[didx.at[j]], add=True)
        return carry

    lax.fori_loop(0, NCHUNK, chunk, 0)
    plsc.subcore_barrier()

    @pl.when(s < NS - 1)
    def _():
        pltpu.sync_copy(acc.at[pl.ds(rbase, RA)], out.at[pl.ds(gbase + rbase, RA)])

    @pl.when(s == NS - 1)
    def _():
        pltpu.sync_copy(acc.at[pl.ds(rbase, RB)], out.at[pl.ds(gbase + rbase, RB)])


@functools.cache
def _make_sc_scatter():
    return pl.kernel(
        _sc_scatter_body,
        out_type=jax.ShapeDtypeStruct((NC * N, HP), jnp.float32),
        mesh=plsc.VectorSubcoreMesh(
            core_axis_name="c", subcore_axis_name="s",
            num_cores=NC, num_subcores=NS,
        ),
        scratch_types=[
            pltpu.VMEM((NCHUNK, CH), jnp.int32),
            pltpu.VMEM((NCHUNK, CH), jnp.int32),
            [pltpu.VMEM((CH, HP), jnp.float32)],
            [pltpu.SemaphoreType.DMA],
            pltpu.SemaphoreType.DMA,
            pltpu.VMEM_SHARED((N + 8, HP), jnp.float32),
        ],
        compiler_params=pltpu.CompilerParams(use_tc_tiling_on_sc=False),
    )


def _sc_scatter(table, src, dst):
    return _make_sc_scatter()(table, src, dst)


def _pad_cols(a):
    if HP == H:
        return a
    return jnp.concatenate(
        [a, jnp.zeros((a.shape[0], HP - H), jnp.float32)], axis=1)


def _proj_body(x_ref, w_ref, o_ref):
    o_ref[...] = _pad_cols(jnp.dot(
        x_ref[0], w_ref[...], preferred_element_type=jnp.float32
    ))[None]


def _mid_body(o1_ref, ba_ref, wb_ref, bb_ref, g_ref, b_ref, wa2_ref, o_ref):
    t = jnp.maximum(o1_ref[0, :, :H] + ba_ref[...], 0.0)
    u = jnp.dot(t, wb_ref[...], preferred_element_type=jnp.float32) + bb_ref[...]
    mu = jnp.mean(u, axis=-1, keepdims=True)
    var = jnp.mean((u - mu) ** 2, axis=-1, keepdims=True)
    v = (u - mu) * lax.rsqrt(var + 1e-5) * g_ref[...] + b_ref[...]
    v = jnp.maximum(v, 0.0)
    o_ref[...] = _pad_cols(
        jnp.dot(v, wa2_ref[...], preferred_element_type=jnp.float32))[None]


def _pool_body(o2_ref, ba_ref, wb_ref, bb_ref, g_ref, b_ref, seg_ref, o_ref, acc):
    j = pl.program_id(1)
    t = jnp.maximum(o2_ref[0, :, :H] + ba_ref[...], 0.0)
    u = jnp.dot(t, wb_ref[...], preferred_element_type=jnp.float32) + bb_ref[...]
    mu = jnp.mean(u, axis=-1, keepdims=True)
    var = jnp.mean((u - mu) ** 2, axis=-1, keepdims=True)
    v = (u - mu) * lax.rsqrt(var + 1e-5) * g_ref[...] + b_ref[...]
    v = jnp.maximum(v, 0.0)
    seg = seg_ref[0, 0]
    gid = lax.broadcasted_iota(jnp.int32, (B, BR), 0).astype(jnp.float32)
    P = (gid == seg[None, :]).astype(jnp.float32)
    vaug = jnp.concatenate([v, jnp.ones((BR, H), jnp.float32)], axis=1)
    blk = jnp.dot(P, vaug, preferred_element_type=jnp.float32)

    @pl.when(j == 0)
    def _():
        acc[...] = blk

    @pl.when(j > 0)
    def _():
        acc[...] = acc[...] + blk

    s = acc[:, :H]
    cnt = acc[:, H : H + 1]
    o_ref[...] = (s + s / jnp.maximum(cnt, 1.0))[None]


def _head_body(comb_ref, w1_ref, b1_ref, w2_ref, b2_ref, w3_ref, b3_ref, o_ref):
    h = jnp.maximum(
        jnp.dot(comb_ref[...], w1_ref[...], preferred_element_type=jnp.float32)
        + b1_ref[...], 0.0)
    h = jnp.maximum(
        jnp.dot(h, w2_ref[...], preferred_element_type=jnp.float32)
        + b2_ref[...], 0.0)
    o_ref[...] = (
        jnp.dot(h, w3_ref[...], preferred_element_type=jnp.float32) + b3_ref[...]
    )


def _row2d(v):
    return v.reshape(1, -1)


def kernel(x1, edge_index1, batch1, x2, edge_index2, batch2, d1, d2,
           c1Wa, c1ba, c1Wb, c1bb, ln1g, ln1b, c2Wa, c2ba, c2Wb, c2bb,
           ln2g, ln2b, fc1W, fc1b, fc2W, fc2b, outW, outb):
    f32 = jnp.float32

    X = jnp.stack([x1, x2])
    pad = ((0, 0), (0, EPTP - EPT))
    src = jnp.stack([
        jnp.pad(edge_index1[0].astype(jnp.int32).reshape(NS, EPT), pad),
        jnp.pad(edge_index2[0].astype(jnp.int32).reshape(NS, EPT), pad) + N,
    ]).reshape(NC, NS, NCHUNK, CH)
    dst = jnp.stack([
        jnp.pad(edge_index1[1].astype(jnp.int32).reshape(NS, EPT), pad,
                constant_values=N),
        jnp.pad(edge_index2[1].astype(jnp.int32).reshape(NS, EPT), pad,
                constant_values=N),
    ]).reshape(NC, NS, NCHUNK, CH)
    segf = jnp.stack([batch1, batch2]).astype(f32).reshape(NC * NBLK, 1, BR)

    xp = pl.pallas_call(
        _proj_body,
        grid=(NC, NBLK),
        in_specs=[
            pl.BlockSpec((1, BR, DF), lambda g, j: (g, j, 0)),
            pl.BlockSpec((DF, H), lambda g, j: (0, 0)),
        ],
        out_specs=pl.BlockSpec((1, BR, HP), lambda g, j: (g, j, 0)),
        out_shape=jax.ShapeDtypeStruct((NC, N, HP), f32),
    )(X, c1Wa)

    o1 = _sc_scatter(xp.reshape(NC * N, HP), src, dst).reshape(NC, N, HP)

    vec_spec = pl.BlockSpec((1, H), lambda g, j: (0, 0))
    mat_spec = pl.BlockSpec((H, H), lambda g, j: (0, 0))
    hp = pl.pallas_call(
        _mid_body,
        grid=(NC, NBLK),
        in_specs=[
            pl.BlockSpec((1, BR, HP), lambda g, j: (g, j, 0)),
            vec_spec, mat_spec, vec_spec, vec_spec, vec_spec, mat_spec,
        ],
        out_specs=pl.BlockSpec((1, BR, HP), lambda g, j: (g, j, 0)),
        out_shape=jax.ShapeDtypeStruct((NC, N, HP), f32),
    )(o1, _row2d(c1ba), c1Wb, _row2d(c1bb), _row2d(ln1g), _row2d(ln1b), c2Wa)

    o2 = _sc_scatter(hp.reshape(NC * N, HP), src, dst).reshape(NC, N, HP)

    emb = pl.pallas_call(
        _pool_body,
        grid=(NC, NBLK),
        in_specs=[
            pl.BlockSpec((1, BR, HP), lambda g, j: (g, j, 0)),
            vec_spec, mat_spec, vec_spec, vec_spec, vec_spec,
            pl.BlockSpec((1, 1, BR), lambda g, j: (g * NBLK + j, 0, 0)),
        ],
        out_specs=pl.BlockSpec((1, B, H), lambda g, j: (g, 0, 0)),
        out_shape=jax.ShapeDtypeStruct((NC, B, H), f32),
        scratch_shapes=[pltpu.VMEM((B, 2 * H), f32)],
    )(o2, _row2d(c2ba), c2Wb, _row2d(c2bb), _row2d(ln2g), _row2d(ln2b), segf)

    comb = jnp.concatenate([emb[0], emb[1], d1, d2], axis=1)
    out = pl.pallas_call(
        _head_body,
        out_shape=jax.ShapeDtypeStruct((B, 1), f32),
    )(comb, fc1W, _row2d(fc1b), fc2W, _row2d(fc2b), outW, _row2d(outb))
    return out


# speedup vs baseline: 1.3206x; 1.3206x over previous
import functools

import jax
import jax.numpy as jnp
from jax import lax
from jax.experimental import pallas as pl
from jax.experimental.pallas import tpu as pltpu
from jax.experimental.pallas import tpu_sc as plsc

N = 10000
E = 320000
B = 64
DF = 128
H = 64

NC = 2
NS = 16
HP = 64
CH = 100
EPT = E // NS
NCHUNK = 200
EPTP = NCHUNK * CH
NBUF = 4
NWIN = 2
WCH = NCHUNK // NWIN
NITER = WCH // NBUF
RA = 632
RB = N - (NS - 1) * RA

BR = 2000
NBLK = N // BR


def _sc_scatter_body(table, src, dst, out, sidx, didx,
                     rows, gsems, ssem, acc):
    c = lax.axis_index("c")
    s = lax.axis_index("s")
    gbase = c * N
    rbase = s * RA

    @pl.when(s < NS - 1)
    def _():
        pltpu.sync_copy(table.at[pl.ds(gbase + rbase, RA)], acc.at[pl.ds(rbase, RA)])

    @pl.when(s == NS - 1)
    def _():
        pltpu.sync_copy(table.at[pl.ds(gbase + rbase, RB)], acc.at[pl.ds(rbase, RB)])

    plsc.subcore_barrier()

    pltpu.sync_copy(src.at[c, s], sidx)
    pltpu.sync_copy(dst.at[c, s], didx)

    def chunk(j, carry):
        pltpu.async_copy(table.at[sidx.at[j]], rows[0], gsems[0]).wait()
        pltpu.sync_copy(rows[0], acc.at[didx.at[j]], add=True)
        return carry

    lax.fori_loop(0, NCHUNK, chunk, 0)
    plsc.subcore_barrier()

    @pl.when(s < NS - 1)
    def _():
        pltpu.sync_copy(acc.at[pl.ds(rbase, RA)], out.at[pl.ds(gbase + rbase, RA)])

    @pl.when(s == NS - 1)
    def _():
        pltpu.sync_copy(acc.at[pl.ds(rbase, RB)], out.at[pl.ds(gbase + rbase, RB)])


@functools.cache
def _make_sc_scatter():
    return pl.kernel(
        _sc_scatter_body,
        out_type=jax.ShapeDtypeStruct((NC * N, HP), jnp.float32),
        mesh=plsc.VectorSubcoreMesh(
            core_axis_name="c", subcore_axis_name="s",
            num_cores=NC, num_subcores=NS,
        ),
        scratch_types=[
            pltpu.VMEM((NCHUNK, CH), jnp.int32),
            pltpu.VMEM((NCHUNK, CH), jnp.int32),
            [pltpu.VMEM((CH, HP), jnp.float32)],
            [pltpu.SemaphoreType.DMA],
            pltpu.SemaphoreType.DMA,
            pltpu.VMEM_SHARED((N + 8, HP), jnp.float32),
        ],
        compiler_params=pltpu.CompilerParams(use_tc_tiling_on_sc=False),
    )


def _sc_scatter(table, src, dst):
    return _make_sc_scatter()(table, src, dst)


def _pad_cols(a):
    if HP == H:
        return a
    return jnp.concatenate(
        [a, jnp.zeros((a.shape[0], HP - H), jnp.float32)], axis=1)


def _proj_body(x_ref, w_ref, o_ref):
    o_ref[...] = _pad_cols(jnp.dot(
        x_ref[0], w_ref[...], preferred_element_type=jnp.float32
    ))[None]


def _mid_body(o1_ref, ba_ref, wb_ref, bb_ref, g_ref, b_ref, wa2_ref, o_ref):
    t = jnp.maximum(o1_ref[0, :, :H] + ba_ref[...], 0.0)
    u = jnp.dot(t, wb_ref[...], preferred_element_type=jnp.float32) + bb_ref[...]
    mu = jnp.mean(u, axis=-1, keepdims=True)
    var = jnp.mean((u - mu) ** 2, axis=-1, keepdims=True)
    v = (u - mu) * lax.rsqrt(var + 1e-5) * g_ref[...] + b_ref[...]
    v = jnp.maximum(v, 0.0)
    o_ref[...] = _pad_cols(
        jnp.dot(v, wa2_ref[...], preferred_element_type=jnp.float32))[None]


def _pool_body(o2_ref, ba_ref, wb_ref, bb_ref, g_ref, b_ref, seg_ref, o_ref, acc):
    j = pl.program_id(1)
    t = jnp.maximum(o2_ref[0, :, :H] + ba_ref[...], 0.0)
    u = jnp.dot(t, wb_ref[...], preferred_element_type=jnp.float32) + bb_ref[...]
    mu = jnp.mean(u, axis=-1, keepdims=True)
    var = jnp.mean((u - mu) ** 2, axis=-1, keepdims=True)
    v = (u - mu) * lax.rsqrt(var + 1e-5) * g_ref[...] + b_ref[...]
    v = jnp.maximum(v, 0.0)
    seg = seg_ref[0, 0]
    gid = lax.broadcasted_iota(jnp.int32, (B, BR), 0).astype(jnp.float32)
    P = (gid == seg[None, :]).astype(jnp.float32)
    vaug = jnp.concatenate([v, jnp.ones((BR, H), jnp.float32)], axis=1)
    blk = jnp.dot(P, vaug, preferred_element_type=jnp.float32)

    @pl.when(j == 0)
    def _():
        acc[...] = blk

    @pl.when(j > 0)
    def _():
        acc[...] = acc[...] + blk

    s = acc[:, :H]
    cnt = acc[:, H : H + 1]
    o_ref[...] = (s + s / jnp.maximum(cnt, 1.0))[None]


def _head_body(comb_ref, w1_ref, b1_ref, w2_ref, b2_ref, w3_ref, b3_ref, o_ref):
    h = jnp.maximum(
        jnp.dot(comb_ref[...], w1_ref[...], preferred_element_type=jnp.float32)
        + b1_ref[...], 0.0)
    h = jnp.maximum(
        jnp.dot(h, w2_ref[...], preferred_element_type=jnp.float32)
        + b2_ref[...], 0.0)
    o_ref[...] = (
        jnp.dot(h, w3_ref[...], preferred_element_type=jnp.float32) + b3_ref[...]
    )


def _row2d(v):
    return v.reshape(1, -1)


def kernel(x1, edge_index1, batch1, x2, edge_index2, batch2, d1, d2,
           c1Wa, c1ba, c1Wb, c1bb, ln1g, ln1b, c2Wa, c2ba, c2Wb, c2bb,
           ln2g, ln2b, fc1W, fc1b, fc2W, fc2b, outW, outb):
    f32 = jnp.float32

    X = jnp.stack([x1, x2])
    pad = ((0, 0), (0, EPTP - EPT))
    src = jnp.stack([
        jnp.pad(edge_index1[0].astype(jnp.int32).reshape(NS, EPT), pad),
        jnp.pad(edge_index2[0].astype(jnp.int32).reshape(NS, EPT), pad) + N,
    ]).reshape(NC, NS, NCHUNK, CH)
    dst = jnp.stack([
        jnp.pad(edge_index1[1].astype(jnp.int32).reshape(NS, EPT), pad,
                constant_values=N),
        jnp.pad(edge_index2[1].astype(jnp.int32).reshape(NS, EPT), pad,
                constant_values=N),
    ]).reshape(NC, NS, NCHUNK, CH)
    segf = jnp.stack([batch1, batch2]).astype(f32).reshape(NC * NBLK, 1, BR)

    xp = pl.pallas_call(
        _proj_body,
        grid=(NC, NBLK),
        in_specs=[
            pl.BlockSpec((1, BR, DF), lambda g, j: (g, j, 0)),
            pl.BlockSpec((DF, H), lambda g, j: (0, 0)),
        ],
        out_specs=pl.BlockSpec((1, BR, HP), lambda g, j: (g, j, 0)),
        out_shape=jax.ShapeDtypeStruct((NC, N, HP), f32),
    )(X, c1Wa)

    o1 = _sc_scatter(xp.reshape(NC * N, HP), src, dst).reshape(NC, N, HP)

    vec_spec = pl.BlockSpec((1, H), lambda g, j: (0, 0))
    mat_spec = pl.BlockSpec((H, H), lambda g, j: (0, 0))
    hp = pl.pallas_call(
        _mid_body,
        grid=(NC, NBLK),
        in_specs=[
            pl.BlockSpec((1, BR, HP), lambda g, j: (g, j, 0)),
            vec_spec, mat_spec, vec_spec, vec_spec, vec_spec, mat_spec,
        ],
        out_specs=pl.BlockSpec((1, BR, HP), lambda g, j: (g, j, 0)),
        out_shape=jax.ShapeDtypeStruct((NC, N, HP), f32),
    )(o1, _row2d(c1ba), c1Wb, _row2d(c1bb), _row2d(ln1g), _row2d(ln1b), c2Wa)

    o2 = _sc_scatter(hp.reshape(NC * N, HP), src, dst).reshape(NC, N, HP)

    emb = pl.pallas_call(
        _pool_body,
        grid=(NC, NBLK),
        in_specs=[
            pl.BlockSpec((1, BR, HP), lambda g, j: (g, j, 0)),
            vec_spec, mat_spec, vec_spec, vec_spec, vec_spec,
            pl.BlockSpec((1, 1, BR), lambda g, j: (g * NBLK + j, 0, 0)),
        ],
        out_specs=pl.BlockSpec((1, B, H), lambda g, j: (g, 0, 0)),
        out_shape=jax.ShapeDtypeStruct((NC, B, H), f32),
        scratch_shapes=[pltpu.VMEM((B, 2 * H), f32)],
    )(o2, _row2d(c2ba), c2Wb, _row2d(c2bb), _row2d(ln2g), _row2d(ln2b), segf)

    comb = jnp.concatenate([emb[0], emb[1], d1, d2], axis=1)
    out = pl.pallas_call(
        _head_body,
        out_shape=jax.ShapeDtypeStruct((B, 1), f32),
    )(comb, fc1W, _row2d(fc1b), fc2W, _row2d(fc2b), outW, _row2d(outb))
    return out


# speedup vs baseline: 2.0132x; 1.5245x over previous
import functools

import jax
import jax.numpy as jnp
from jax import lax
from jax.experimental import pallas as pl
from jax.experimental.pallas import tpu as pltpu
from jax.experimental.pallas import tpu_sc as plsc

N = 10000
E = 320000
B = 64
DF = 128
H = 64

NC = 2
NS = 16
HP = 64
CH = 100
EPT = E // NS
NCHUNK = 200
EPTP = NCHUNK * CH
NBUF = 4
NWIN = 2
WCH = NCHUNK // NWIN
NITER = WCH // NBUF
RA = 632
RB = N - (NS - 1) * RA

BR = 2000
NBLK = N // BR


def _sc_scatter_body(table, src, dst, out, sidx, didx,
                     rows, gsems, ssem, acc):
    c = lax.axis_index("c")
    s = lax.axis_index("s")
    gbase = c * N
    rbase = s * RA

    @pl.when(s < NS - 1)
    def _():
        pltpu.sync_copy(table.at[pl.ds(gbase + rbase, RA)], acc.at[pl.ds(rbase, RA)])

    @pl.when(s == NS - 1)
    def _():
        pltpu.sync_copy(table.at[pl.ds(gbase + rbase, RB)], acc.at[pl.ds(rbase, RB)])

    plsc.subcore_barrier()

    pltpu.sync_copy(src.at[c, s], sidx)
    pltpu.sync_copy(dst.at[c, s], didx)

    def wait_gather(j, buf, sem):
        pltpu.make_async_copy(table.at[sidx.at[j]], buf, sem).wait()

    pltpu.async_copy(table.at[sidx.at[0]], rows[0], gsems[0])
    wait_gather(0, rows[0], gsems[0])
    pltpu.async_copy(table.at[sidx.at[1]], rows[1], gsems[1])
    pltpu.sync_copy(rows[0], acc.at[didx.at[0]], add=True)

    def pair(k, carry):
        jb = 2 * k + 1
        ja = 2 * k + 2
        pltpu.async_copy(table.at[sidx.at[ja]], rows[0], gsems[0])
        wait_gather(jb, rows[1], gsems[1])
        pltpu.sync_copy(rows[1], acc.at[didx.at[jb]], add=True)
        pltpu.async_copy(table.at[sidx.at[ja + 1]], rows[1], gsems[1])
        wait_gather(ja, rows[0], gsems[0])
        pltpu.sync_copy(rows[0], acc.at[didx.at[ja]], add=True)
        return carry

    lax.fori_loop(0, NCHUNK // 2 - 1, pair, 0)
    wait_gather(NCHUNK - 1, rows[1], gsems[1])
    pltpu.sync_copy(rows[1], acc.at[didx.at[NCHUNK - 1]], add=True)
    plsc.subcore_barrier()

    @pl.when(s < NS - 1)
    def _():
        pltpu.sync_copy(acc.at[pl.ds(rbase, RA)], out.at[pl.ds(gbase + rbase, RA)])

    @pl.when(s == NS - 1)
    def _():
        pltpu.sync_copy(acc.at[pl.ds(rbase, RB)], out.at[pl.ds(gbase + rbase, RB)])


@functools.cache
def _make_sc_scatter():
    return pl.kernel(
        _sc_scatter_body,
        out_type=jax.ShapeDtypeStruct((NC * N, HP), jnp.float32),
        mesh=plsc.VectorSubcoreMesh(
            core_axis_name="c", subcore_axis_name="s",
            num_cores=NC, num_subcores=NS,
        ),
        scratch_types=[
            pltpu.VMEM((NCHUNK, CH), jnp.int32),
            pltpu.VMEM((NCHUNK, CH), jnp.int32),
            [pltpu.VMEM((CH, HP), jnp.float32) for _ in range(2)],
            [pltpu.SemaphoreType.DMA for _ in range(2)],
            pltpu.SemaphoreType.DMA,
            pltpu.VMEM_SHARED((N + 8, HP), jnp.float32),
        ],
        compiler_params=pltpu.CompilerParams(use_tc_tiling_on_sc=False),
    )


def _sc_scatter(table, src, dst):
    return _make_sc_scatter()(table, src, dst)


def _pad_cols(a):
    if HP == H:
        return a
    return jnp.concatenate(
        [a, jnp.zeros((a.shape[0], HP - H), jnp.float32)], axis=1)


def _proj_body(x_ref, w_ref, o_ref):
    o_ref[...] = _pad_cols(jnp.dot(
        x_ref[0], w_ref[...], preferred_element_type=jnp.float32
    ))[None]


def _mid_body(o1_ref, ba_ref, wb_ref, bb_ref, g_ref, b_ref, wa2_ref, o_ref):
    t = jnp.maximum(o1_ref[0, :, :H] + ba_ref[...], 0.0)
    u = jnp.dot(t, wb_ref[...], preferred_element_type=jnp.float32) + bb_ref[...]
    mu = jnp.mean(u, axis=-1, keepdims=True)
    var = jnp.mean((u - mu) ** 2, axis=-1, keepdims=True)
    v = (u - mu) * lax.rsqrt(var + 1e-5) * g_ref[...] + b_ref[...]
    v = jnp.maximum(v, 0.0)
    o_ref[...] = _pad_cols(
        jnp.dot(v, wa2_ref[...], preferred_element_type=jnp.float32))[None]


def _pool_body(o2_ref, ba_ref, wb_ref, bb_ref, g_ref, b_ref, seg_ref, o_ref, acc):
    j = pl.program_id(1)
    t = jnp.maximum(o2_ref[0, :, :H] + ba_ref[...], 0.0)
    u = jnp.dot(t, wb_ref[...], preferred_element_type=jnp.float32) + bb_ref[...]
    mu = jnp.mean(u, axis=-1, keepdims=True)
    var = jnp.mean((u - mu) ** 2, axis=-1, keepdims=True)
    v = (u - mu) * lax.rsqrt(var + 1e-5) * g_ref[...] + b_ref[...]
    v = jnp.maximum(v, 0.0)
    seg = seg_ref[0, 0]
    gid = lax.broadcasted_iota(jnp.int32, (B, BR), 0).astype(jnp.float32)
    P = (gid == seg[None, :]).astype(jnp.float32)
    vaug = jnp.concatenate([v, jnp.ones((BR, H), jnp.float32)], axis=1)
    blk = jnp.dot(P, vaug, preferred_element_type=jnp.float32)

    @pl.when(j == 0)
    def _():
        acc[...] = blk

    @pl.when(j > 0)
    def _():
        acc[...] = acc[...] + blk

    s = acc[:, :H]
    cnt = acc[:, H : H + 1]
    o_ref[...] = (s + s / jnp.maximum(cnt, 1.0))[None]


def _head_body(comb_ref, w1_ref, b1_ref, w2_ref, b2_ref, w3_ref, b3_ref, o_ref):
    h = jnp.maximum(
        jnp.dot(comb_ref[...], w1_ref[...], preferred_element_type=jnp.float32)
        + b1_ref[...], 0.0)
    h = jnp.maximum(
        jnp.dot(h, w2_ref[...], preferred_element_type=jnp.float32)
        + b2_ref[...], 0.0)
    o_ref[...] = (
        jnp.dot(h, w3_ref[...], preferred_element_type=jnp.float32) + b3_ref[...]
    )


def _row2d(v):
    return v.reshape(1, -1)


def kernel(x1, edge_index1, batch1, x2, edge_index2, batch2, d1, d2,
           c1Wa, c1ba, c1Wb, c1bb, ln1g, ln1b, c2Wa, c2ba, c2Wb, c2bb,
           ln2g, ln2b, fc1W, fc1b, fc2W, fc2b, outW, outb):
    f32 = jnp.float32

    X = jnp.stack([x1, x2])
    pad = ((0, 0), (0, EPTP - EPT))
    src = jnp.stack([
        jnp.pad(edge_index1[0].astype(jnp.int32).reshape(NS, EPT), pad),
        jnp.pad(edge_index2[0].astype(jnp.int32).reshape(NS, EPT), pad) + N,
    ]).reshape(NC, NS, NCHUNK, CH)
    dst = jnp.stack([
        jnp.pad(edge_index1[1].astype(jnp.int32).reshape(NS, EPT), pad,
                constant_values=N),
        jnp.pad(edge_index2[1].astype(jnp.int32).reshape(NS, EPT), pad,
                constant_values=N),
    ]).reshape(NC, NS, NCHUNK, CH)
    segf = jnp.stack([batch1, batch2]).astype(f32).reshape(NC * NBLK, 1, BR)

    xp = pl.pallas_call(
        _proj_body,
        grid=(NC, NBLK),
        in_specs=[
            pl.BlockSpec((1, BR, DF), lambda g, j: (g, j, 0)),
            pl.BlockSpec((DF, H), lambda g, j: (0, 0)),
        ],
        out_specs=pl.BlockSpec((1, BR, HP), lambda g, j: (g, j, 0)),
        out_shape=jax.ShapeDtypeStruct((NC, N, HP), f32),
    )(X, c1Wa)

    o1 = _sc_scatter(xp.reshape(NC * N, HP), src, dst).reshape(NC, N, HP)

    vec_spec = pl.BlockSpec((1, H), lambda g, j: (0, 0))
    mat_spec = pl.BlockSpec((H, H), lambda g, j: (0, 0))
    hp = pl.pallas_call(
        _mid_body,
        grid=(NC, NBLK),
        in_specs=[
            pl.BlockSpec((1, BR, HP), lambda g, j: (g, j, 0)),
            vec_spec, mat_spec, vec_spec, vec_spec, vec_spec, mat_spec,
        ],
        out_specs=pl.BlockSpec((1, BR, HP), lambda g, j: (g, j, 0)),
        out_shape=jax.ShapeDtypeStruct((NC, N, HP), f32),
    )(o1, _row2d(c1ba), c1Wb, _row2d(c1bb), _row2d(ln1g), _row2d(ln1b), c2Wa)

    o2 = _sc_scatter(hp.reshape(NC * N, HP), src, dst).reshape(NC, N, HP)

    emb = pl.pallas_call(
        _pool_body,
        grid=(NC, NBLK),
        in_specs=[
            pl.BlockSpec((1, BR, HP), lambda g, j: (g, j, 0)),
            vec_spec, mat_spec, vec_spec, vec_spec, vec_spec,
            pl.BlockSpec((1, 1, BR), lambda g, j: (g * NBLK + j, 0, 0)),
        ],
        out_specs=pl.BlockSpec((1, B, H), lambda g, j: (g, 0, 0)),
        out_shape=jax.ShapeDtypeStruct((NC, B, H), f32),
        scratch_shapes=[pltpu.VMEM((B, 2 * H), f32)],
    )(o2, _row2d(c2ba), c2Wb, _row2d(c2bb), _row2d(ln2g), _row2d(ln2b), segf)

    comb = jnp.concatenate([emb[0], emb[1], d1, d2], axis=1)
    out = pl.pallas_call(
        _head_body,
        out_shape=jax.ShapeDtypeStruct((B, 1), f32),
    )(comb, fc1W, _row2d(fc1b), fc2W, _row2d(fc2b), outW, _row2d(outb))
    return out
